# baseline (speedup 1.0000x reference)
import functools

import jax
import jax.numpy as jnp
from jax import lax
from jax.experimental import pallas as pl
from jax.experimental.pallas import tpu as pltpu
from jax.experimental.pallas import tpu_sc as plsc

N = 10000
E = 320000
D = 128

NC = 2
NS = 16
DH = D // NC
C = 128
CH = -(-E // (NS * C))
EPAD = NS * CH * C - E
NPAD = 10240
ROWS_PER_TILE = NPAD // NS


def _sc_aggregate_body(xh_hbm, src_hbm, dst_hbm, agg_out, deg_out,
                       src_v, dst_v, rows0_v, rows1_v, rows2_v, rows3_v,
                       ones_v, zeros_v, agg_sh, deg_sh,
                       sem0, sem1, sem2, sem3):
    cid = lax.axis_index("c")
    sid = lax.axis_index("s")

    pltpu.sync_copy(src_hbm.at[sid], src_v)
    pltpu.sync_copy(dst_hbm.at[sid], dst_v)

    zeros16 = jnp.zeros((16,), jnp.float32)
    ones16 = jnp.ones((16,), jnp.float32)

    def fill_row(r, _):
        def fill_col(k, _):
            rows0_v[r, pl.ds(k * 16, 16)] = zeros16
            return 0
        lax.fori_loop(0, DH // 16, fill_col, 0)
        ones_v[r, pl.ds(0, 16)] = ones16
        zeros_v[r, pl.ds(0, 16)] = zeros16
        return 0
    lax.fori_loop(0, C, fill_row, 0)

    for j in range(ROWS_PER_TILE // C):
        pltpu.sync_copy(rows0_v, agg_sh.at[pl.ds(sid * ROWS_PER_TILE + j * C, C)])
        pltpu.sync_copy(zeros_v, deg_sh.at[pl.ds(sid * ROWS_PER_TILE + j * C, C)])
    plsc.subcore_barrier()

    bufs = (rows0_v, rows1_v, rows2_v, rows3_v)
    sems = (sem0, sem1, sem2, sem3)

    xv = xh_hbm.at[cid]

    def _gather(j, b):
        pltpu.async_copy(xv.at[src_v.at[j]], bufs[b], sems[b])

    def _wait(j, b):
        pltpu.make_async_copy(xv.at[src_v.at[j]], bufs[b],
                              sems[b]).wait()

    def _process(j, b):
        _wait(j, b)
        pltpu.sync_copy(bufs[b], agg_sh.at[dst_v.at[j]], add=True)

        @pl.when(cid == (j % 2))
        def _deg():
            pltpu.sync_copy(ones_v, deg_sh.at[dst_v.at[j]], add=True)

    for b in range(3):
        _gather(b, b)

    def quad(q, _):
        for b in range(4):
            j = 4 * q + b
            jn = j + 3

            @pl.when(jn < CH)
            def _fire():
                _gather(jn, (b + 3) % 4)
            _wait(j, b)
            pltpu.sync_copy(bufs[b], agg_sh.at[dst_v.at[j]], add=True)

            @pl.when(cid == (b % 2))
            def _deg():
                pltpu.sync_copy(ones_v, deg_sh.at[dst_v.at[j]], add=True)
        return 0
    lax.fori_loop(0, CH // 4, quad, 0)
    for t in range(CH % 4):
        j = (CH // 4) * 4 + t
        _process(j, j % 4)

    plsc.subcore_barrier()

    pltpu.sync_copy(agg_sh.at[pl.ds(sid * ROWS_PER_TILE, ROWS_PER_TILE)],
                    agg_out.at[cid, pl.ds(sid * ROWS_PER_TILE, ROWS_PER_TILE)])
    pltpu.sync_copy(deg_sh.at[pl.ds(sid * ROWS_PER_TILE, ROWS_PER_TILE)],
                    deg_out.at[cid, pl.ds(sid * ROWS_PER_TILE, ROWS_PER_TILE)])


_sc_aggregate = functools.partial(
    pl.kernel,
    out_type=(jax.ShapeDtypeStruct((NC, NPAD, DH), jnp.float32),
              jax.ShapeDtypeStruct((NC, NPAD, 16), jnp.float32)),
    mesh=plsc.VectorSubcoreMesh(core_axis_name="c", subcore_axis_name="s",
                                num_cores=NC, num_subcores=NS),
    scratch_types=[
        pltpu.VMEM((CH, C), jnp.int32),
        pltpu.VMEM((CH, C), jnp.int32),
        pltpu.VMEM((C, DH), jnp.float32),
        pltpu.VMEM((C, DH), jnp.float32),
        pltpu.VMEM((C, DH), jnp.float32),
        pltpu.VMEM((C, DH), jnp.float32),
        pltpu.VMEM((C, 16), jnp.float32),
        pltpu.VMEM((C, 16), jnp.float32),
        pltpu.VMEM_SHARED((NPAD, DH), jnp.float32),
        pltpu.VMEM_SHARED((NPAD, 16), jnp.float32),
        pltpu.SemaphoreType.DMA,
        pltpu.SemaphoreType.DMA,
        pltpu.SemaphoreType.DMA,
        pltpu.SemaphoreType.DMA,
    ],
    compiler_params=pltpu.CompilerParams(use_tc_tiling_on_sc=False),
)(_sc_aggregate_body)


R = 2000
GSTEPS = N // R


def _tc_hr_body(x_ref, wr_ref, bl_ref, o_ref):
    dn = (((1,), (1,)), ((), ()))
    o_ref[...] = (lax.dot_general(x_ref[...], wr_ref[...], dn,
                                  precision=lax.Precision.HIGHEST,
                                  preferred_element_type=jnp.float32)
                  + bl_ref[...][None, :])


_tc_hr = pl.pallas_call(
    _tc_hr_body,
    grid=(GSTEPS,),
    in_specs=[
        pl.BlockSpec((R, D), lambda i: (i, 0)),
        pl.BlockSpec((D, D), lambda i: (0, 0)),
        pl.BlockSpec((D,), lambda i: (0,)),
    ],
    out_specs=pl.BlockSpec((R, D), lambda i: (i, 0)),
    out_shape=jax.ShapeDtypeStruct((N, D), jnp.float32),
)


def _tc_finish_body(aggp_ref, degp_ref, hr_ref, x_ref, wl_ref,
                    g_ref, b_ref, o_ref, h_scr, st_scr):
    i = pl.program_id(0)
    blk = jnp.where(i < GSTEPS, i, i - GSTEPS)
    row0 = pl.multiple_of(blk * R, R)

    @pl.when(i < GSTEPS)
    def _phase_h():
        agg = jnp.concatenate([aggp_ref[0], aggp_ref[1]], axis=1)
        deg = (degp_ref[0] + degp_ref[1])[:, 0:1]
        mean_agg = agg * (1.0 / jnp.maximum(deg, 1.0))
        dn = (((1,), (1,)), ((), ()))
        h = (lax.dot_general(mean_agg, wl_ref[...], dn,
                             precision=lax.Precision.HIGHEST,
                             preferred_element_type=jnp.float32)
             + hr_ref[...])
        h_scr[pl.ds(row0, R), :] = h
        s1 = jnp.sum(h, axis=0, keepdims=True)
        s2 = jnp.sum(h * h, axis=0, keepdims=True)
        part = jnp.concatenate(
            [s1, s2, jnp.zeros((6, D), jnp.float32)], axis=0)

        @pl.when(i == 0)
        def _init():
            st_scr[...] = part

        @pl.when(i > 0)
        def _acc():
            st_scr[...] += part
        o_ref[...] = h

    @pl.when(i >= GSTEPS)
    def _phase_norm():
        h = h_scr[pl.ds(row0, R), :]
        mu = st_scr[0:1, :] * (1.0 / N)
        var = st_scr[1:2, :] * (1.0 / N) - mu * mu
        hn = ((h - mu) * lax.rsqrt(var + 1e-5) * g_ref[...][None, :]
              + b_ref[...][None, :])
        o_ref[...] = jnp.maximum(hn, 0.0) + x_ref[...]


_tc_finish = pl.pallas_call(
    _tc_finish_body,
    grid=(2 * GSTEPS,),
    in_specs=[
        pl.BlockSpec((NC, R, DH), lambda i: (0, jnp.minimum(i, GSTEPS - 1), 0)),
        pl.BlockSpec((NC, R, 16), lambda i: (0, jnp.minimum(i, GSTEPS - 1), 0)),
        pl.BlockSpec((R, D), lambda i: (jnp.minimum(i, GSTEPS - 1), 0)),
        pl.BlockSpec((R, D), lambda i: (jnp.maximum(i, GSTEPS) - GSTEPS, 0)),
        pl.BlockSpec((D, D), lambda i: (0, 0)),
        pl.BlockSpec((D,), lambda i: (0,)),
        pl.BlockSpec((D,), lambda i: (0,)),
    ],
    out_specs=pl.BlockSpec((R, D), lambda i: (jnp.maximum(i, GSTEPS) - GSTEPS, 0)),
    out_shape=jax.ShapeDtypeStruct((N, D), jnp.float32),
    scratch_shapes=[
        pltpu.VMEM((N, D), jnp.float32),
        pltpu.VMEM((8, D), jnp.float32),
    ],
)


def kernel(x, edge_index, W_l, b_l, W_r, gamma, beta):
    src = jnp.concatenate(
        [edge_index[0].astype(jnp.int32), jnp.zeros((EPAD,), jnp.int32)])
    dst = jnp.concatenate(
        [edge_index[1].astype(jnp.int32),
         N + (jnp.arange(EPAD, dtype=jnp.int32) % (NPAD - N))])
    src = src.reshape(NS, CH, C)
    dst = dst.reshape(NS, CH, C)
    xh = x.reshape(N, NC, DH).transpose(1, 0, 2)
    aggp, degp = _sc_aggregate(xh, src, dst)
    hr = _tc_hr(x, W_r, b_l)
    return _tc_finish(aggp, degp, hr, x, W_l, gamma, beta)


# speedup vs baseline: 1.2323x; 1.2323x over previous
import functools

import jax
import jax.numpy as jnp
from jax import lax
from jax.experimental import pallas as pl
from jax.experimental.pallas import tpu as pltpu
from jax.experimental.pallas import tpu_sc as plsc

N = 10000
E = 320000
D = 128

NC = 2
NS = 16
DH = D // NC
C = 80
CH = -(-E // (NS * C))
EPAD = NS * CH * C - E
NPAD = 10240
DEGW = 8
ROWS_PER_TILE = NPAD // NS


NB = 8


def _sc_aggregate_body(xh_hbm, src_hbm, dst_hbm, agg_out, deg_out,
                       src_v, dst_v, bufs, ones_v, zeros_v, agg_sh, deg_sh,
                       gsems, ssems, dsem):
    cid = lax.axis_index("c")
    sid = lax.axis_index("s")

    pltpu.sync_copy(src_hbm.at[sid], src_v)
    pltpu.sync_copy(dst_hbm.at[sid], dst_v)

    zeros16 = jnp.zeros((16,), jnp.float32)
    ones16 = jnp.ones((16,), jnp.float32)

    def fill_row(r, _):
        def fill_col(k, _):
            bufs[0][r, pl.ds(k * 16, 16)] = zeros16
            return 0
        lax.fori_loop(0, DH // 16, fill_col, 0)
        return 0
    lax.fori_loop(0, C, fill_row, 0)

    lanes = lax.iota(jnp.int32, 16)

    def fill_deg(i, _):
        flat = i * 16 + lanes
        ridx = lax.shift_right_logical(flat, 3)
        cidx = lax.bitwise_and(flat, 7)
        plsc.store_scatter(ones_v, [ridx, cidx], ones16)
        plsc.store_scatter(zeros_v, [ridx, cidx], zeros16)
        return 0
    lax.fori_loop(0, C * DEGW // 16, fill_deg, 0)

    for j in range(ROWS_PER_TILE // C):
        pltpu.sync_copy(bufs[0], agg_sh.at[pl.ds(sid * ROWS_PER_TILE + j * C, C)])
        pltpu.sync_copy(zeros_v, deg_sh.at[pl.ds(sid * ROWS_PER_TILE + j * C, C)])
    plsc.subcore_barrier()

    xv = xh_hbm.at[cid]

    def _gather(j, b):
        pltpu.async_copy(xv.at[src_v.at[j]], bufs[b], gsems[b])

    def _wait_gather(j, b):
        pltpu.make_async_copy(xv.at[src_v.at[j]], bufs[b], gsems[b]).wait()

    def _scatter(j, b):
        pltpu.async_copy(bufs[b], agg_sh.at[dst_v.at[j]], ssems[b], add=True)

    def _wait_scatter(b):
        pltpu.make_async_copy(bufs[b], agg_sh.at[dst_v.at[0]],
                              ssems[b]).wait()

    def _deg_fire(j):
        pltpu.async_copy(ones_v, deg_sh.at[dst_v.at[j]], dsem, add=True)

    def _deg_wait():
        pltpu.make_async_copy(ones_v, deg_sh.at[dst_v.at[0]], dsem).wait()

    FA = NB // 2
    for b in range(FA):
        _gather(b, b)

    def octet(q, _):
        for b in range(NB):
            j = NB * q + b
            jn = j + FA
            bn = (b + FA) % NB

            @pl.when(jn < CH)
            def _fire():
                @pl.when(j >= FA)
                def _drain_prev():
                    _wait_scatter(bn)
                _gather(jn, bn)
            _wait_gather(j, b)
            _scatter(j, b)

            @pl.when(cid == (b % 2))
            def _deg():
                @pl.when(j >= 2)
                def _drain_deg():
                    _deg_wait()
                _deg_fire(j)
        return 0
    lax.fori_loop(0, CH // NB, octet, 0)
    for t in range(CH % NB):
        j = (CH // NB) * NB + t
        b = j % NB
        _wait_gather(j, b)
        _scatter(j, b)

        @pl.when(cid == (j % 2))
        def _deg_tail():
            _deg_wait()
            _deg_fire(j)

    for b in range(NB):
        _wait_scatter(b)
    _deg_wait()

    plsc.subcore_barrier()

    pltpu.sync_copy(agg_sh.at[pl.ds(sid * ROWS_PER_TILE, ROWS_PER_TILE)],
                    agg_out.at[cid, pl.ds(sid * ROWS_PER_TILE, ROWS_PER_TILE)])
    pltpu.sync_copy(deg_sh.at[pl.ds(sid * ROWS_PER_TILE, ROWS_PER_TILE)],
                    deg_out.at[cid, pl.ds(sid * ROWS_PER_TILE, ROWS_PER_TILE)])


_sc_aggregate = functools.partial(
    pl.kernel,
    out_type=(jax.ShapeDtypeStruct((NC, NPAD, DH), jnp.float32),
              jax.ShapeDtypeStruct((NC, NPAD, DEGW), jnp.float32)),
    mesh=plsc.VectorSubcoreMesh(core_axis_name="c", subcore_axis_name="s",
                                num_cores=NC, num_subcores=NS),
    scratch_types=[
        pltpu.VMEM((CH, C), jnp.int32),
        pltpu.VMEM((CH, C), jnp.int32),
        tuple(pltpu.VMEM((C, DH), jnp.float32) for _ in range(NB)),
        pltpu.VMEM((C, DEGW), jnp.float32),
        pltpu.VMEM((C, DEGW), jnp.float32),
        pltpu.VMEM_SHARED((NPAD, DH), jnp.float32),
        pltpu.VMEM_SHARED((NPAD, DEGW), jnp.float32),
        tuple(pltpu.SemaphoreType.DMA for _ in range(NB)),
        tuple(pltpu.SemaphoreType.DMA for _ in range(NB)),
        pltpu.SemaphoreType.DMA,
    ],
    compiler_params=pltpu.CompilerParams(use_tc_tiling_on_sc=False,
                                         needs_layout_passes=False),
)(_sc_aggregate_body)


R = 2000
GSTEPS = N // R


def _tc_hr_body(x_ref, wr_ref, bl_ref, o_ref):
    dn = (((1,), (1,)), ((), ()))
    o_ref[...] = (lax.dot_general(x_ref[...], wr_ref[...], dn,
                                  precision=lax.Precision.HIGHEST,
                                  preferred_element_type=jnp.float32)
                  + bl_ref[...][None, :])


_tc_hr = pl.pallas_call(
    _tc_hr_body,
    grid=(GSTEPS,),
    in_specs=[
        pl.BlockSpec((R, D), lambda i: (i, 0)),
        pl.BlockSpec((D, D), lambda i: (0, 0)),
        pl.BlockSpec((D,), lambda i: (0,)),
    ],
    out_specs=pl.BlockSpec((R, D), lambda i: (i, 0)),
    out_shape=jax.ShapeDtypeStruct((N, D), jnp.float32),
)


def _tc_finish_body(aggp_ref, degp_ref, hr_ref, x_ref, wl_ref,
                    g_ref, b_ref, o_ref, h_scr, st_scr):
    i = pl.program_id(0)
    blk = jnp.where(i < GSTEPS, i, i - GSTEPS)
    row0 = pl.multiple_of(blk * R, R)

    @pl.when(i < GSTEPS)
    def _phase_h():
        agg = jnp.concatenate([aggp_ref[0], aggp_ref[1]], axis=1)
        deg = (degp_ref[0] + degp_ref[1])[:, 0:1]
        mean_agg = agg * (1.0 / jnp.maximum(deg, 1.0))
        dn = (((1,), (1,)), ((), ()))
        h = (lax.dot_general(mean_agg, wl_ref[...], dn,
                             precision=lax.Precision.HIGHEST,
                             preferred_element_type=jnp.float32)
             + hr_ref[...])
        h_scr[pl.ds(row0, R), :] = h
        s1 = jnp.sum(h, axis=0, keepdims=True)
        s2 = jnp.sum(h * h, axis=0, keepdims=True)
        part = jnp.concatenate(
            [s1, s2, jnp.zeros((6, D), jnp.float32)], axis=0)

        @pl.when(i == 0)
        def _init():
            st_scr[...] = part

        @pl.when(i > 0)
        def _acc():
            st_scr[...] += part
        o_ref[...] = h

    @pl.when(i >= GSTEPS)
    def _phase_norm():
        h = h_scr[pl.ds(row0, R), :]
        mu = st_scr[0:1, :] * (1.0 / N)
        var = st_scr[1:2, :] * (1.0 / N) - mu * mu
        hn = ((h - mu) * lax.rsqrt(var + 1e-5) * g_ref[...][None, :]
              + b_ref[...][None, :])
        o_ref[...] = jnp.maximum(hn, 0.0) + x_ref[...]


_tc_finish = pl.pallas_call(
    _tc_finish_body,
    grid=(2 * GSTEPS,),
    in_specs=[
        pl.BlockSpec((NC, R, DH), lambda i: (0, jnp.minimum(i, GSTEPS - 1), 0)),
        pl.BlockSpec((NC, R, DEGW), lambda i: (0, jnp.minimum(i, GSTEPS - 1), 0)),
        pl.BlockSpec((R, D), lambda i: (jnp.minimum(i, GSTEPS - 1), 0)),
        pl.BlockSpec((R, D), lambda i: (jnp.maximum(i, GSTEPS) - GSTEPS, 0)),
        pl.BlockSpec((D, D), lambda i: (0, 0)),
        pl.BlockSpec((D,), lambda i: (0,)),
        pl.BlockSpec((D,), lambda i: (0,)),
    ],
    out_specs=pl.BlockSpec((R, D), lambda i: (jnp.maximum(i, GSTEPS) - GSTEPS, 0)),
    out_shape=jax.ShapeDtypeStruct((N, D), jnp.float32),
    scratch_shapes=[
        pltpu.VMEM((N, D), jnp.float32),
        pltpu.VMEM((8, D), jnp.float32),
    ],
)


def kernel(x, edge_index, W_l, b_l, W_r, gamma, beta):
    src = jnp.concatenate(
        [edge_index[0].astype(jnp.int32), jnp.zeros((EPAD,), jnp.int32)])
    dst = jnp.concatenate(
        [edge_index[1].astype(jnp.int32),
         N + (jnp.arange(EPAD, dtype=jnp.int32) % (NPAD - N))])
    src = src.reshape(NS, CH, C)
    dst = dst.reshape(NS, CH, C)
    xh = x.reshape(N, NC, DH).transpose(1, 0, 2)
    aggp, degp = _sc_aggregate(xh, src, dst)
    hr = _tc_hr(x, W_r, b_l)
    return _tc_finish(aggp, degp, hr, x, W_l, gamma, beta)
